# baseline (speedup 1.0000x reference)
import functools

import jax
import jax.numpy as jnp
from jax import lax
from jax.experimental import pallas as pl
from jax.experimental.pallas import tpu as pltpu
from jax.experimental.pallas import tpu_sc as plsc

_BETA = 0.25
_TB = 8


def _pq_step(zn_ref, ze_ref, cb_ref, tok_ref, loss_ref, util_ref):
    t = pl.program_id(0)
    TB, B, D = ze_ref.shape
    K = cb_ref.shape[1]

    @pl.when(t == 0)
    def _init():
        loss_ref[:, :] = jnp.zeros((1, 1), jnp.float32)
        util_ref[:, :] = jnp.zeros((1, 1), jnp.float32)

    ze_all = ze_ref[:]
    cb_all = cb_ref[:]
    zn_all = zn_ref[:, 0, :]

    scores = jnp.stack([
        jax.lax.dot_general(ze_all[i], cb_all[i],
                            dimension_numbers=(((1,), (1,)), ((), ())))
        for i in range(TB)
    ])

    cbn = jnp.sum(cb_all * cb_all, axis=-1)
    dist = (zn_all[:, :, None] - 2.0 * scores) + cbn[:, None, :]
    m = jnp.min(dist, axis=-1, keepdims=True)
    mask = dist == m
    iota_k = jax.lax.broadcasted_iota(jnp.int32, (TB, B, K), 2)
    k_idx = jnp.min(jnp.where(mask, iota_k, K), axis=-1).astype(jnp.int32)
    tok_ref[:, 0, :] = k_idx

    loss_ref[:, :] = loss_ref[:, :] + jnp.sum(m)
    used = jnp.max(mask.astype(jnp.float32), axis=1)
    util_ref[:, :] = util_ref[:, :] + jnp.sum(used)


def _sc_gather(table, idx, n_rows, d):
    info = plsc.get_sparse_core_info()
    nc, ns = info.num_cores, info.num_subcores
    nw = nc * ns
    per_w = n_rows // nw
    mesh = plsc.VectorSubcoreMesh(core_axis_name="c", subcore_axis_name="s")

    @functools.partial(
        pl.kernel, mesh=mesh,
        out_type=jax.ShapeDtypeStruct((n_rows, d), jnp.float32),
        scratch_types=[
            pltpu.VMEM((per_w,), jnp.int32),
            pltpu.VMEM((per_w, d), jnp.float32),
            pltpu.SemaphoreType.DMA,
        ],
    )
    def gather_kernel(table_hbm, idx_hbm, out_hbm, idx_v, rows_v, sem):
        wid = lax.axis_index("s") * nc + lax.axis_index("c")
        base = wid * per_w
        pltpu.sync_copy(idx_hbm.at[pl.ds(base, per_w)], idx_v)
        pltpu.async_copy(table_hbm.at[idx_v], rows_v, sem).wait()
        pltpu.sync_copy(rows_v, out_hbm.at[pl.ds(base, per_w)])

    return gather_kernel(table, idx)


def kernel(z_e, codebooks):
    B, T, D = z_e.shape
    K = codebooks.shape[1]
    ze_t = jnp.transpose(z_e, (1, 0, 2))
    zn_t = jnp.sum(ze_t ** 2, axis=-1).reshape(T, 1, B)
    tok_t, loss, util = pl.pallas_call(
        _pq_step,
        grid=(T // _TB,),
        in_specs=[
            pl.BlockSpec((_TB, 1, B), lambda t: (t, 0, 0)),
            pl.BlockSpec((_TB, B, D), lambda t: (t, 0, 0)),
            pl.BlockSpec((_TB, K, D), lambda t: (t, 0, 0)),
        ],
        out_specs=[
            pl.BlockSpec((_TB, 1, B), lambda t: (t, 0, 0)),
            pl.BlockSpec((1, 1), lambda t: (0, 0)),
            pl.BlockSpec((1, 1), lambda t: (0, 0)),
        ],
        out_shape=[
            jax.ShapeDtypeStruct((T, 1, B), jnp.int32),
            jax.ShapeDtypeStruct((1, 1), jnp.float32),
            jax.ShapeDtypeStruct((1, 1), jnp.float32),
        ],
    )(zn_t, ze_t, codebooks)
    tokens = jnp.transpose(tok_t[:, 0, :], (1, 0))

    flat_idx = (tokens + jnp.arange(T, dtype=jnp.int32)[None, :] * K).reshape(-1)
    wide = _sc_gather(codebooks.reshape(T * K // 2, 2 * D),
                      jnp.right_shift(flat_idx, 1), B * T, 2 * D)
    zq = jnp.where((flat_idx & 1)[:, None] == 1, wide[:, D:], wide[:, :D])
    zq = zq.reshape(B, T, D)

    z_q_st = z_e + (zq - z_e)
    vq_loss = _BETA * (loss[0, 0] / jnp.float32(T * B * D))
    utilization = util[0, 0] / jnp.float32(T * K)
    return z_q_st, tokens, vq_loss, utilization


# speedup vs baseline: 1.0276x; 1.0276x over previous
import functools

import jax
import jax.numpy as jnp
from jax import lax
from jax.experimental import pallas as pl
from jax.experimental.pallas import tpu as pltpu
from jax.experimental.pallas import tpu_sc as plsc

_BETA = 0.25
_TB = 32


def _pq_step(zn_ref, ze_ref, cb_ref, tok_ref, loss_ref, util_ref):
    t = pl.program_id(0)
    TB, B, D = ze_ref.shape
    K = cb_ref.shape[1]

    @pl.when(t == 0)
    def _init():
        loss_ref[:, :] = jnp.zeros((1, 1), jnp.float32)
        util_ref[:, :] = jnp.zeros((1, 1), jnp.float32)

    ze_all = ze_ref[:]
    cb_all = cb_ref[:]
    zn_all = zn_ref[:, 0, :]

    scores = jnp.stack([
        jax.lax.dot_general(ze_all[i], cb_all[i],
                            dimension_numbers=(((1,), (1,)), ((), ())))
        for i in range(TB)
    ])

    cbn = jnp.sum(cb_all * cb_all, axis=-1)
    dist = (zn_all[:, :, None] - 2.0 * scores) + cbn[:, None, :]
    m = jnp.min(dist, axis=-1, keepdims=True)
    mask = dist == m
    iota_k = jax.lax.broadcasted_iota(jnp.int32, (TB, B, K), 2)
    k_idx = jnp.min(jnp.where(mask, iota_k, K), axis=-1).astype(jnp.int32)
    tok_ref[:, 0, :] = k_idx

    loss_ref[:, :] = loss_ref[:, :] + jnp.sum(m)
    used = jnp.max(mask.astype(jnp.float32), axis=1)
    util_ref[:, :] = util_ref[:, :] + jnp.sum(used)


def _sc_gather(table, idx, n_rows, d):
    info = plsc.get_sparse_core_info()
    nc, ns = info.num_cores, info.num_subcores
    nw = nc * ns
    per_w = n_rows // nw
    mesh = plsc.VectorSubcoreMesh(core_axis_name="c", subcore_axis_name="s")

    @functools.partial(
        pl.kernel, mesh=mesh,
        out_type=jax.ShapeDtypeStruct((n_rows, d), jnp.float32),
        scratch_types=[
            pltpu.VMEM((per_w,), jnp.int32),
            pltpu.VMEM((per_w, d), jnp.float32),
            pltpu.SemaphoreType.DMA,
        ],
    )
    def gather_kernel(table_hbm, idx_hbm, out_hbm, idx_v, rows_v, sem):
        wid = lax.axis_index("s") * nc + lax.axis_index("c")
        base = wid * per_w
        pltpu.sync_copy(idx_hbm.at[pl.ds(base, per_w)], idx_v)
        pltpu.async_copy(table_hbm.at[idx_v], rows_v, sem).wait()
        pltpu.sync_copy(rows_v, out_hbm.at[pl.ds(base, per_w)])

    return gather_kernel(table, idx)


def kernel(z_e, codebooks):
    B, T, D = z_e.shape
    K = codebooks.shape[1]
    ze_t = jnp.transpose(z_e, (1, 0, 2))
    zn_t = jnp.sum(ze_t ** 2, axis=-1).reshape(T, 1, B)
    tok_t, loss, util = pl.pallas_call(
        _pq_step,
        grid=(T // _TB,),
        in_specs=[
            pl.BlockSpec((_TB, 1, B), lambda t: (t, 0, 0)),
            pl.BlockSpec((_TB, B, D), lambda t: (t, 0, 0)),
            pl.BlockSpec((_TB, K, D), lambda t: (t, 0, 0)),
        ],
        out_specs=[
            pl.BlockSpec((_TB, 1, B), lambda t: (t, 0, 0)),
            pl.BlockSpec((1, 1), lambda t: (0, 0)),
            pl.BlockSpec((1, 1), lambda t: (0, 0)),
        ],
        out_shape=[
            jax.ShapeDtypeStruct((T, 1, B), jnp.int32),
            jax.ShapeDtypeStruct((1, 1), jnp.float32),
            jax.ShapeDtypeStruct((1, 1), jnp.float32),
        ],
    )(zn_t, ze_t, codebooks)
    tokens = jnp.transpose(tok_t[:, 0, :], (1, 0))

    flat_idx = (tokens + jnp.arange(T, dtype=jnp.int32)[None, :] * K).reshape(-1)
    wide = _sc_gather(codebooks.reshape(T * K // 2, 2 * D),
                      jnp.right_shift(flat_idx, 1), B * T, 2 * D)
    zq = jnp.where((flat_idx & 1)[:, None] == 1, wide[:, D:], wide[:, :D])
    zq = zq.reshape(B, T, D)

    z_q_st = z_e + (zq - z_e)
    vq_loss = _BETA * (loss[0, 0] / jnp.float32(T * B * D))
    utilization = util[0, 0] / jnp.float32(T * K)
    return z_q_st, tokens, vq_loss, utilization


# speedup vs baseline: 1.3505x; 1.3142x over previous
import jax
import jax.numpy as jnp
from jax.experimental import pallas as pl

_BETA = 0.25
_TB = 8


def _pq_step(ze_ref, cb_ref, zq_ref, tok_ref, loss_ref, util_ref):
    t = pl.program_id(0)
    TB, B, D = ze_ref.shape
    K = cb_ref.shape[1]

    @pl.when(t == 0)
    def _init():
        loss_ref[:, :] = jnp.zeros((1, 1), jnp.float32)
        util_ref[:, :] = jnp.zeros((1, 1), jnp.float32)

    ze_all = ze_ref[:]
    cb_all = cb_ref[:]
    zn_all = jnp.sum(ze_all * ze_all, axis=-1)

    scores = jnp.stack([
        jax.lax.dot_general(ze_all[i], cb_all[i],
                            dimension_numbers=(((1,), (1,)), ((), ())))
        for i in range(TB)
    ])

    cbn = jnp.sum(cb_all * cb_all, axis=-1)
    dist = (zn_all[:, :, None] - 2.0 * scores) + cbn[:, None, :]
    m = jnp.min(dist, axis=-1, keepdims=True)
    iota_k = jax.lax.broadcasted_iota(jnp.int32, (TB, B, K), 2)
    k_idx = jnp.min(jnp.where(dist == m, iota_k, K), axis=-1).astype(jnp.int32)
    onehot = (k_idx[:, :, None] == iota_k).astype(jnp.float32)

    zq = jnp.stack([
        jax.lax.dot_general(onehot[i], cb_all[i],
                            dimension_numbers=(((1,), (0,)), ((), ())),
                            precision=jax.lax.Precision.HIGHEST)
        for i in range(TB)
    ])

    zq_ref[:] = jnp.swapaxes(ze_all + (zq - ze_all), 0, 1)
    tok_ref[:, 0, :] = k_idx
    d = ze_all - zq
    loss_ref[:, :] = loss_ref[:, :] + jnp.sum(d * d)
    util_ref[:, :] = util_ref[:, :] + jnp.sum(jnp.max(onehot, axis=1))


def kernel(z_e, codebooks):
    B, T, D = z_e.shape
    K = codebooks.shape[1]
    ze_t = jnp.transpose(z_e, (1, 0, 2))
    z_q_st, tok_t, loss, util = pl.pallas_call(
        _pq_step,
        grid=(T // _TB,),
        in_specs=[
            pl.BlockSpec((_TB, B, D), lambda t: (t, 0, 0)),
            pl.BlockSpec((_TB, K, D), lambda t: (t, 0, 0)),
        ],
        out_specs=[
            pl.BlockSpec((B, _TB, D), lambda t: (0, t, 0)),
            pl.BlockSpec((_TB, 1, B), lambda t: (t, 0, 0)),
            pl.BlockSpec((1, 1), lambda t: (0, 0)),
            pl.BlockSpec((1, 1), lambda t: (0, 0)),
        ],
        out_shape=[
            jax.ShapeDtypeStruct((B, T, D), jnp.float32),
            jax.ShapeDtypeStruct((T, 1, B), jnp.int32),
            jax.ShapeDtypeStruct((1, 1), jnp.float32),
            jax.ShapeDtypeStruct((1, 1), jnp.float32),
        ],
    )(ze_t, codebooks)
    tokens = jnp.transpose(tok_t[:, 0, :], (1, 0))
    vq_loss = _BETA * (loss[0, 0] / jnp.float32(T * B * D))
    utilization = util[0, 0] / jnp.float32(T * K)
    return z_q_st, tokens, vq_loss, utilization


# speedup vs baseline: 1.7845x; 1.3214x over previous
import jax
import jax.numpy as jnp
from jax.experimental import pallas as pl

_BETA = 0.25
_TB = 8


def _pq_step(ze_ref, cb_ref, zq_ref, tok_ref, loss_ref, util_ref):
    t = pl.program_id(0)
    TB, B, D = ze_ref.shape
    K = cb_ref.shape[1]

    @pl.when(t == 0)
    def _init():
        loss_ref[:, :] = jnp.zeros((1, 1), jnp.float32)
        util_ref[:, :] = jnp.zeros((1, 1), jnp.float32)

    ze_all = ze_ref[:]
    cb_all = cb_ref[:]
    zn_all = jnp.sum(ze_all * ze_all, axis=-1)

    scores = jnp.stack([
        jax.lax.dot_general(ze_all[i], cb_all[i],
                            dimension_numbers=(((1,), (1,)), ((), ())))
        for i in range(TB)
    ])

    cbn = jnp.sum(cb_all * cb_all, axis=-1)
    dist = (zn_all[:, :, None] - 2.0 * scores) + cbn[:, None, :]
    m = jnp.min(dist, axis=-1, keepdims=True)
    iota_k = jax.lax.broadcasted_iota(jnp.int32, (TB, B, K), 2)
    k_idx = jnp.min(jnp.where(dist == m, iota_k, K), axis=-1).astype(jnp.int32)
    onehot = (k_idx[:, :, None] == iota_k).astype(jnp.float32)

    zq = jnp.stack([
        jax.lax.dot_general(onehot[i], cb_all[i],
                            dimension_numbers=(((1,), (0,)), ((), ())))
        for i in range(TB)
    ])

    zq_ref[:] = jnp.swapaxes(ze_all + (zq - ze_all), 0, 1)
    tok_ref[:, 0, :] = k_idx
    d = ze_all - zq
    loss_ref[:, :] = loss_ref[:, :] + jnp.sum(d * d)
    util_ref[:, :] = util_ref[:, :] + jnp.sum(jnp.max(onehot, axis=1))


def kernel(z_e, codebooks):
    B, T, D = z_e.shape
    K = codebooks.shape[1]
    ze_t = jnp.transpose(z_e, (1, 0, 2))
    z_q_st, tok_t, loss, util = pl.pallas_call(
        _pq_step,
        grid=(T // _TB,),
        in_specs=[
            pl.BlockSpec((_TB, B, D), lambda t: (t, 0, 0)),
            pl.BlockSpec((_TB, K, D), lambda t: (t, 0, 0)),
        ],
        out_specs=[
            pl.BlockSpec((B, _TB, D), lambda t: (0, t, 0)),
            pl.BlockSpec((_TB, 1, B), lambda t: (t, 0, 0)),
            pl.BlockSpec((1, 1), lambda t: (0, 0)),
            pl.BlockSpec((1, 1), lambda t: (0, 0)),
        ],
        out_shape=[
            jax.ShapeDtypeStruct((B, T, D), jnp.float32),
            jax.ShapeDtypeStruct((T, 1, B), jnp.int32),
            jax.ShapeDtypeStruct((1, 1), jnp.float32),
            jax.ShapeDtypeStruct((1, 1), jnp.float32),
        ],
    )(ze_t, codebooks)
    tokens = jnp.transpose(tok_t[:, 0, :], (1, 0))
    vq_loss = _BETA * (loss[0, 0] / jnp.float32(T * B * D))
    utilization = util[0, 0] / jnp.float32(T * K)
    return z_q_st, tokens, vq_loss, utilization


# speedup vs baseline: 1.8105x; 1.0146x over previous
import jax
import jax.numpy as jnp
from jax.experimental import pallas as pl

_BETA = 0.25
_TB = 8


def _pq_step(ze_ref, cb_ref, zq_ref, tok_ref, loss_ref, util_ref):
    t = pl.program_id(0)
    B, TB, D = ze_ref.shape
    K = cb_ref.shape[1]

    @pl.when(t == 0)
    def _init():
        loss_ref[:, :] = jnp.zeros((1, 1), jnp.float32)
        util_ref[:, :] = jnp.zeros((1, 1), jnp.float32)

    ze_all = jnp.swapaxes(ze_ref[:], 0, 1)
    cb_all = cb_ref[:]
    zn_all = jnp.sum(ze_all * ze_all, axis=-1)

    scores = jnp.stack([
        jax.lax.dot_general(ze_all[i], cb_all[i],
                            dimension_numbers=(((1,), (1,)), ((), ())))
        for i in range(TB)
    ])

    cbn = jnp.sum(cb_all * cb_all, axis=-1)
    dist = (zn_all[:, :, None] - 2.0 * scores) + cbn[:, None, :]
    m = jnp.min(dist, axis=-1, keepdims=True)
    iota_k = jax.lax.broadcasted_iota(jnp.int32, (TB, B, K), 2)
    k_idx = jnp.min(jnp.where(dist == m, iota_k, K), axis=-1).astype(jnp.int32)
    onehot = (k_idx[:, :, None] == iota_k).astype(jnp.float32)

    zq = jnp.stack([
        jax.lax.dot_general(onehot[i], cb_all[i],
                            dimension_numbers=(((1,), (0,)), ((), ())))
        for i in range(TB)
    ])

    zq_ref[:] = jnp.swapaxes(ze_all + (zq - ze_all), 0, 1)
    tok_ref[:, 0, :] = k_idx
    d = ze_all - zq
    loss_ref[:, :] = loss_ref[:, :] + jnp.sum(d * d)
    util_ref[:, :] = util_ref[:, :] + jnp.sum(jnp.max(onehot, axis=1))


def kernel(z_e, codebooks):
    B, T, D = z_e.shape
    K = codebooks.shape[1]
    z_q_st, tok_t, loss, util = pl.pallas_call(
        _pq_step,
        grid=(T // _TB,),
        in_specs=[
            pl.BlockSpec((B, _TB, D), lambda t: (0, t, 0)),
            pl.BlockSpec((_TB, K, D), lambda t: (t, 0, 0)),
        ],
        out_specs=[
            pl.BlockSpec((B, _TB, D), lambda t: (0, t, 0)),
            pl.BlockSpec((_TB, 1, B), lambda t: (t, 0, 0)),
            pl.BlockSpec((1, 1), lambda t: (0, 0)),
            pl.BlockSpec((1, 1), lambda t: (0, 0)),
        ],
        out_shape=[
            jax.ShapeDtypeStruct((B, T, D), jnp.float32),
            jax.ShapeDtypeStruct((T, 1, B), jnp.int32),
            jax.ShapeDtypeStruct((1, 1), jnp.float32),
            jax.ShapeDtypeStruct((1, 1), jnp.float32),
        ],
    )(z_e, codebooks)
    tokens = jnp.transpose(tok_t[:, 0, :], (1, 0))
    vq_loss = _BETA * (loss[0, 0] / jnp.float32(T * B * D))
    utilization = util[0, 0] / jnp.float32(T * K)
    return z_q_st, tokens, vq_loss, utilization


# speedup vs baseline: 1.8379x; 1.0151x over previous
import jax
import jax.numpy as jnp
from jax.experimental import pallas as pl

_BETA = 0.25
_TB = 8


def _pq_step(ze_ref, cb_ref, zq_ref, tok_ref, loss_ref, util_ref):
    t = pl.program_id(0)
    B, TB, D = ze_ref.shape
    K = cb_ref.shape[1]

    @pl.when(t == 0)
    def _init():
        loss_ref[:, :] = jnp.zeros((1, 1), jnp.float32)
        util_ref[:, :] = jnp.zeros((1, 1), jnp.float32)

    ze_all = jnp.swapaxes(ze_ref[:], 0, 1)
    cb_all = cb_ref[:]
    zn_all = jnp.sum(ze_all * ze_all, axis=-1)

    scores = jnp.stack([
        jax.lax.dot_general(ze_all[i], cb_all[i],
                            dimension_numbers=(((1,), (1,)), ((), ())))
        for i in range(TB)
    ])

    cbn = jnp.sum(cb_all * cb_all, axis=-1)
    dist = (zn_all[:, :, None] - 2.0 * scores) + cbn[:, None, :]
    m = jnp.min(dist, axis=-1, keepdims=True)
    iota_k = jax.lax.broadcasted_iota(jnp.int32, (TB, B, K), 2)
    k_idx = jnp.min(jnp.where(dist == m, iota_k, K), axis=-1).astype(jnp.int32)
    onehot = (k_idx[:, :, None] == iota_k).astype(jnp.float32)

    zq = jnp.stack([
        jax.lax.dot_general(onehot[i], cb_all[i],
                            dimension_numbers=(((1,), (0,)), ((), ())))
        for i in range(TB)
    ])

    zq_ref[:] = jnp.swapaxes(zq, 0, 1)
    tok_ref[:, 0, :] = k_idx
    loss_ref[:, :] = loss_ref[:, :] + jnp.sum(m)
    util_ref[:, :] = util_ref[:, :] + jnp.sum(jnp.max(onehot, axis=1))


def kernel(z_e, codebooks):
    B, T, D = z_e.shape
    K = codebooks.shape[1]
    z_q_st, tok_t, loss, util = pl.pallas_call(
        _pq_step,
        grid=(T // _TB,),
        in_specs=[
            pl.BlockSpec((B, _TB, D), lambda t: (0, t, 0)),
            pl.BlockSpec((_TB, K, D), lambda t: (t, 0, 0)),
        ],
        out_specs=[
            pl.BlockSpec((B, _TB, D), lambda t: (0, t, 0)),
            pl.BlockSpec((_TB, 1, B), lambda t: (t, 0, 0)),
            pl.BlockSpec((1, 1), lambda t: (0, 0)),
            pl.BlockSpec((1, 1), lambda t: (0, 0)),
        ],
        out_shape=[
            jax.ShapeDtypeStruct((B, T, D), jnp.float32),
            jax.ShapeDtypeStruct((T, 1, B), jnp.int32),
            jax.ShapeDtypeStruct((1, 1), jnp.float32),
            jax.ShapeDtypeStruct((1, 1), jnp.float32),
        ],
    )(z_e, codebooks)
    tokens = jnp.transpose(tok_t[:, 0, :], (1, 0))
    vq_loss = _BETA * (loss[0, 0] / jnp.float32(T * B * D))
    utilization = util[0, 0] / jnp.float32(T * K)
    return z_q_st, tokens, vq_loss, utilization


# speedup vs baseline: 1.9506x; 1.0613x over previous
import jax
import jax.numpy as jnp
from jax.experimental import pallas as pl

_BETA = 0.25
_TB = 16


def _pq_step(ze_ref, cb_ref, zq_ref, tok_ref, loss_ref, util_ref):
    t = pl.program_id(0)
    B, TB, D = ze_ref.shape
    K = cb_ref.shape[1]

    @pl.when(t == 0)
    def _init():
        loss_ref[:, :] = jnp.zeros((1, 1), jnp.float32)
        util_ref[:, :] = jnp.zeros((1, 1), jnp.float32)

    ze_all = jnp.swapaxes(ze_ref[:], 0, 1)
    cb_all = cb_ref[:]
    zn_all = jnp.sum(ze_all * ze_all, axis=-1)

    scores = jnp.stack([
        jax.lax.dot_general(ze_all[i], cb_all[i],
                            dimension_numbers=(((1,), (1,)), ((), ())))
        for i in range(TB)
    ])

    cbn = jnp.sum(cb_all * cb_all, axis=-1)
    dist = (zn_all[:, :, None] - 2.0 * scores) + cbn[:, None, :]
    m = jnp.min(dist, axis=-1, keepdims=True)
    iota_k = jax.lax.broadcasted_iota(jnp.int32, (TB, B, K), 2)
    k_idx = jnp.min(jnp.where(dist == m, iota_k, K), axis=-1).astype(jnp.int32)
    onehot = (k_idx[:, :, None] == iota_k).astype(jnp.float32)

    zq = jnp.stack([
        jax.lax.dot_general(onehot[i], cb_all[i],
                            dimension_numbers=(((1,), (0,)), ((), ())))
        for i in range(TB)
    ])

    zq_ref[:] = jnp.swapaxes(zq, 0, 1)
    tok_ref[:, 0, :] = k_idx
    loss_ref[:, :] = loss_ref[:, :] + jnp.sum(m)
    util_ref[:, :] = util_ref[:, :] + jnp.sum(jnp.max(onehot, axis=1))


def kernel(z_e, codebooks):
    B, T, D = z_e.shape
    K = codebooks.shape[1]
    z_q_st, tok_t, loss, util = pl.pallas_call(
        _pq_step,
        grid=(T // _TB,),
        in_specs=[
            pl.BlockSpec((B, _TB, D), lambda t: (0, t, 0)),
            pl.BlockSpec((_TB, K, D), lambda t: (t, 0, 0)),
        ],
        out_specs=[
            pl.BlockSpec((B, _TB, D), lambda t: (0, t, 0)),
            pl.BlockSpec((_TB, 1, B), lambda t: (t, 0, 0)),
            pl.BlockSpec((1, 1), lambda t: (0, 0)),
            pl.BlockSpec((1, 1), lambda t: (0, 0)),
        ],
        out_shape=[
            jax.ShapeDtypeStruct((B, T, D), jnp.float32),
            jax.ShapeDtypeStruct((T, 1, B), jnp.int32),
            jax.ShapeDtypeStruct((1, 1), jnp.float32),
            jax.ShapeDtypeStruct((1, 1), jnp.float32),
        ],
    )(z_e, codebooks)
    tokens = jnp.transpose(tok_t[:, 0, :], (1, 0))
    vq_loss = _BETA * (loss[0, 0] / jnp.float32(T * B * D))
    utilization = util[0, 0] / jnp.float32(T * K)
    return z_q_st, tokens, vq_loss, utilization
